# Initial kernel scaffold; baseline (speedup 1.0000x reference)
import jax
import jax.numpy as jnp
from jax.experimental import pallas as pl


def kernel(x, emb, base_emb):
    raise NotImplementedError("write your pallas kernel here")


# speedup vs baseline: 1.5822x; 1.5822x over previous
import functools

import jax
import jax.numpy as jnp
from jax import lax
from jax.experimental import pallas as pl
from jax.experimental.pallas import tpu as pltpu
from jax.experimental.pallas import tpu_sc as plsc

_K = 6
_KS = 245
_NUM_CTX = sum(5 ** i for i in range(_K + 1))
_B, _L = 4096, 50
_NCOL = _L + 1
_R = _B * _NCOL

_NC, _NS = 2, 16
_NW = _NC * _NS
_ROWS_W = _R // _NW
_CH = 96
_NCHUNK = _ROWS_W // _CH
_KSP = 256


def _inds_body(x_ref, out_ref):
    x = x_ref[:]
    offs = [(5 ** m - 1) // 4 for m in range(_K + 1)]
    cols = []
    v = jnp.zeros((_B, 1), jnp.int32)
    cols.append(v + offs[0])
    for i in range(1, _K):
        v = v * 5 + x[:, i - 1:i]
        cols.append(v + offs[i])
    wide = _L - _K + 1
    big = jnp.zeros((_B, wide), jnp.int32)
    for j in range(_K):
        big = big * 5 + x[:, j:j + wide]
    cols.append(big + offs[_K])
    out_ref[:] = jnp.concatenate(cols, axis=1)


def _compute_inds(x):
    return pl.pallas_call(
        _inds_body,
        out_shape=jax.ShapeDtypeStruct((_B, _NCOL), jnp.int32),
    )(x)


def _bias_body(e_ref, b_ref, o_ref):
    o_ref[:, : _KS] = e_ref[:] + b_ref[:]
    o_ref[:, _KS:] = jnp.zeros((o_ref.shape[0], _KSP - _KS), jnp.float32)


def _bias_table(emb, base_emb):
    rb = 1024
    grid = (_NUM_CTX + rb - 1) // rb
    return pl.pallas_call(
        _bias_body,
        grid=(grid,),
        in_specs=[
            pl.BlockSpec((rb, _KS), lambda i: (i, 0)),
            pl.BlockSpec((1, _KS), lambda i: (0, 0)),
        ],
        out_specs=pl.BlockSpec((rb, _KSP), lambda i: (i, 0)),
        out_shape=jax.ShapeDtypeStruct((_NUM_CTX, _KSP), jnp.float32),
    )(emb, base_emb.reshape(1, _KS))


def _sc_gather_body(tab_hbm, idx_hbm, out_hbm, idx_v, buf, sem):
    wid = lax.axis_index("s") * _NC + lax.axis_index("c")
    base = wid * _ROWS_W
    pltpu.sync_copy(idx_hbm.at[pl.ds(base, _ROWS_W)], idx_v)

    def body(ch, carry):
        idx_slice = idx_v.at[pl.ds(ch * _CH, _CH)]
        pltpu.async_copy(tab_hbm.at[idx_slice], buf, sem).wait()
        pltpu.sync_copy(buf, out_hbm.at[pl.ds(base + ch * _CH, _CH)])
        return carry

    lax.fori_loop(0, _NCHUNK, body, 0)


_sc_gather = functools.partial(
    pl.kernel,
    mesh=plsc.VectorSubcoreMesh(core_axis_name="c", subcore_axis_name="s"),
    out_type=jax.ShapeDtypeStruct((_R, _KSP), jnp.float32),
    scratch_types=[
        pltpu.VMEM((_ROWS_W,), jnp.int32),
        pltpu.VMEM((_CH, _KSP), jnp.float32),
        pltpu.SemaphoreType.DMA,
    ],
)(_sc_gather_body)


def kernel(x, emb, base_emb):
    x = x.astype(jnp.int32)
    inds = _compute_inds(x)
    emb_b = _bias_table(emb, base_emb)
    out = _sc_gather(emb_b, inds.reshape(_R))
    return out[:, : _KS].reshape(_B, _NCOL, _KS)
